# baseline (speedup 1.0000x reference)
import functools

import jax
import jax.numpy as jnp
import numpy as np
from jax import lax
from jax.experimental import pallas as pl
from jax.experimental.pallas import tpu as pltpu
from jax.experimental.pallas import tpu_sc as plsc

NUM_SERIES = 64
K = 8
B = 4096
T = 200
D = 32
BR = 512


@functools.cache
def _make_sc_gather():
    info = plsc.get_sparse_core_info()
    nc, ns = info.num_cores, info.num_subcores
    nw = nc * ns
    b_per_w = B // nw
    mesh = plsc.VectorSubcoreMesh(core_axis_name="c", subcore_axis_name="s")

    @functools.partial(
        pl.kernel,
        mesh=mesh,
        out_type=jax.ShapeDtypeStruct((B, D), jnp.float32),
        scratch_types=[
            pltpu.VMEM((b_per_w,), jnp.int32),
            pltpu.VMEM((b_per_w, D), jnp.float32),
            pltpu.SemaphoreType.DMA,
        ],
        compiler_params=pltpu.CompilerParams(use_tc_tiling_on_sc=False),
    )
    def gather_kernel(table_hbm, idx_hbm, out_hbm, idx_v, rows_v, sem):
        wid = lax.axis_index("s") * nc + lax.axis_index("c")
        base = wid * b_per_w
        pltpu.sync_copy(idx_hbm.at[pl.ds(base, b_per_w)], idx_v)
        pltpu.async_copy(table_hbm.at[idx_v], rows_v, sem).wait()
        pltpu.sync_copy(rows_v, out_hbm.at[pl.ds(base, b_per_w)])

    return gather_kernel


_COS_COEFS = (-59.58028076034274, 61.107297158754044,
              -19.552735135991313, 0.998566776846633)


_CH = 32


def _fourier_body(x_ref, g_ref, o_ref):
    def chunk(i, _):
        rows = pl.ds(i * _CH, _CH)
        xb = x_ref[rows, :]
        g = g_ref[rows, :]
        acc = jnp.broadcast_to(g[:, 3 * K:3 * K + 1], xb.shape)
        for k in range(K):
            f = g[:, k:k + 1]
            ph = g[:, K + k:K + k + 1]
            amp = g[:, 2 * K + k:2 * K + k + 1]
            u = xb * f + ph
            r = u - jnp.round(u)
            s = r * r
            p = _COS_COEFS[0]
            for c in _COS_COEFS[1:]:
                p = p * s + c
            acc = acc + amp * p
        o_ref[rows, :] = acc
        return 0

    lax.fori_loop(0, BR // _CH, chunk, 0)


def _tc_fourier(x, g):
    return pl.pallas_call(
        _fourier_body,
        grid=(B // BR,),
        in_specs=[
            pl.BlockSpec((BR, T), lambda i: (i, 0)),
            pl.BlockSpec((BR, D), lambda i: (i, 0)),
        ],
        out_specs=pl.BlockSpec((BR, T), lambda i: (i, 0)),
        out_shape=jax.ShapeDtypeStruct((B, T), jnp.float32),
        compiler_params=pltpu.CompilerParams(
            dimension_semantics=("arbitrary",),
        ),
    )(x, g)


def kernel(x, series_id, layers1_weight, layers1_bias, layers2_weight, layers2_bias):
    w1 = layers1_weight.reshape(NUM_SERIES, K)
    b1 = (1.0 / (2.0 * np.pi)) * layers1_bias.reshape(NUM_SERIES, K)
    w2 = layers2_weight.reshape(NUM_SERIES, K)
    b2 = layers2_bias.reshape(NUM_SERIES, 1)
    table = jnp.concatenate(
        [w1, b1, w2, b2, jnp.zeros((NUM_SERIES, D - 3 * K - 1), jnp.float32)],
        axis=1,
    )
    sid = series_id.reshape(-1).astype(jnp.int32)
    g = _make_sc_gather()(table, sid)
    return _tc_fourier(x, g)


# speedup vs baseline: 1.1505x; 1.1505x over previous
import functools

import jax
import jax.numpy as jnp
import numpy as np
from jax import lax
from jax.experimental import pallas as pl
from jax.experimental.pallas import tpu as pltpu
from jax.experimental.pallas import tpu_sc as plsc

NUM_SERIES = 64
K = 8
B = 4096
T = 200
D = 128
BR = 512


@functools.cache
def _make_sc_gather():
    info = plsc.get_sparse_core_info()
    nc, ns = info.num_cores, info.num_subcores
    nw = nc * ns
    mesh = plsc.VectorSubcoreMesh(core_axis_name="c", subcore_axis_name="s")

    @functools.partial(
        pl.kernel,
        mesh=mesh,
        out_type=jax.ShapeDtypeStruct((B, D), jnp.float32),
        scratch_types=[
            pltpu.VMEM((B // nw,), jnp.int32),
            pltpu.VMEM((B // nw, D), jnp.float32),
            pltpu.SemaphoreType.DMA,
        ],
        compiler_params=pltpu.CompilerParams(use_tc_tiling_on_sc=False),
    )
    def gather_kernel(table_hbm, idx_hbm, out_hbm, idx_v, rows_v, sem):
        bw = B // nw
        wid = lax.axis_index("s") * nc + lax.axis_index("c")
        base = wid * bw
        pltpu.sync_copy(idx_hbm.at[pl.ds(base, bw)], idx_v)
        pltpu.async_copy(table_hbm.at[idx_v], rows_v, sem).wait()
        pltpu.sync_copy(rows_v, out_hbm.at[pl.ds(base, bw)])

    return gather_kernel


_COS_COEFS = (-59.58028076034274, 61.107297158754044,
              -19.552735135991313, 0.998566776846633)


def _fourier_body(x_ref, g_ref, o_ref):
    xb = x_ref[...]
    g = g_ref[...]
    acc = jnp.broadcast_to(g[:, 3 * K:3 * K + 1], xb.shape)
    for k in range(K):
        f = g[:, k:k + 1]
        ph = g[:, K + k:K + k + 1]
        amp = g[:, 2 * K + k:2 * K + k + 1]
        u = xb * f + ph
        r = u - jnp.round(u)
        s = r * r
        p = _COS_COEFS[0]
        for c in _COS_COEFS[1:]:
            p = p * s + c
        acc = acc + amp * p
    o_ref[...] = acc


def _tc_fourier(x, g):
    rows = x.shape[0]
    return pl.pallas_call(
        _fourier_body,
        grid=(rows // BR,),
        in_specs=[
            pl.BlockSpec((BR, T), lambda i: (i, 0)),
            pl.BlockSpec((BR, D), lambda i: (i, 0)),
        ],
        out_specs=pl.BlockSpec((BR, T), lambda i: (i, 0)),
        out_shape=jax.ShapeDtypeStruct((rows, T), jnp.float32),
        compiler_params=pltpu.CompilerParams(
            dimension_semantics=("arbitrary",),
        ),
    )(x, g)


def kernel(x, series_id, layers1_weight, layers1_bias, layers2_weight, layers2_bias):
    t1 = jnp.pad(layers1_weight[:, 0, :], ((0, 0), (0, D - K)))
    t2 = jnp.pad((1.0 / (2.0 * np.pi)) * layers1_bias[:, 0, :],
                 ((0, 0), (K, D - 2 * K)))
    t3 = jnp.pad(layers2_weight[:, :, 0], ((0, 0), (2 * K, D - 3 * K)))
    t4 = jnp.pad(layers2_bias[:, 0, :], ((0, 0), (3 * K, D - 3 * K - 1)))
    table = t1 + t2 + t3 + t4
    sid = series_id.reshape(-1).astype(jnp.int32)
    g = _make_sc_gather()(table, sid)
    return _tc_fourier(x, g)
